# baseline (device time: 1347458 ns/iter reference)
import jax
import jax.numpy as jnp
from jax import lax
from jax.experimental import pallas as pl
from jax.experimental.pallas import tpu as pltpu

N_Y = 4
SC = 512
EB = 1024
N_EB = 8
R = 2048
RH = R // 2


def kernel(O, Wo):
    B, S, Hs, D = O.shape
    K = Hs * D
    E = Wo.shape[1]

    O16 = O.reshape(B, S, K).astype(jnp.bfloat16)
    Oc = O16.reshape(B, N_Y, SC, K).swapaxes(0, 1).reshape(N_Y, B * SC, K)

    def body(oc_ref, w_ref, out_ref, send_ref, comm_ref,
             ostage_ref, wstage_ref, wstage32_ref, outstage_ref,
             send_sems, recv_sems, credit_sems, o_sem, w_sem):
        my_x = lax.axis_index("x")
        my_y = lax.axis_index("y")
        my_z = lax.axis_index("z")
        right = (my_x, (my_y + 1) % N_Y, my_z)
        left = (my_x, (my_y - 1) % N_Y, my_z)

        def signal_credit(slot):
            pl.semaphore_signal(
                credit_sems.at[slot], inc=1,
                device_id=left, device_id_type=pl.DeviceIdType.MESH)

        def ring_rdma(sslot, cslot):
            return pltpu.make_async_remote_copy(
                src_ref=send_ref.at[sslot],
                dst_ref=comm_ref.at[cslot],
                send_sem=send_sems.at[sslot],
                recv_sem=recv_sems.at[cslot],
                device_id=right,
                device_id_type=pl.DeviceIdType.MESH,
            )

        def load_w(eb):
            cw = pltpu.make_async_copy(
                w_ref.at[:, pl.ds(eb * EB, EB)], wstage32_ref, w_sem)
            cw.start()
            cw.wait()
            for h in range(2):
                rows = pl.ds(h * RH, RH)
                wstage_ref[rows] = wstage32_ref[rows].astype(jnp.bfloat16)

        def partial_into_send(sslot, chunk):
            co = pltpu.make_async_copy(oc_ref.at[chunk], ostage_ref, o_sem)
            co.start()
            co.wait()
            for h in range(2):
                rows = pl.ds(h * RH, RH)
                send_ref[sslot, rows] = jnp.dot(
                    ostage_ref[rows, :], wstage_ref[...],
                    preferred_element_type=jnp.float32,
                ).astype(jnp.bfloat16)

        def accumulate_send(sslot, cslot):
            for h in range(2):
                rows = pl.ds(h * RH, RH)
                send_ref[sslot, rows] = (send_ref[sslot, rows]
                                         + comm_ref[cslot, rows])

        load_w(0)
        partial_into_send(0, (my_y - 1) % N_Y)

        def eb_body(eb, carry):
            ecols = pl.ds(eb * EB, EB)

            @pl.when(eb > 0)
            def _():
                pl.semaphore_wait(credit_sems.at[0], 1)
            rd0 = ring_rdma(0, 0)
            rd0.start()

            partial_into_send(1, (my_y - 2) % N_Y)
            rd0.wait_recv()

            accumulate_send(1, 0)
            signal_credit(0)

            @pl.when(eb > 0)
            def _():
                pl.semaphore_wait(credit_sems.at[1], 1)
            rd1 = ring_rdma(1, 1)
            rd1.start()

            @pl.when(eb > 0)
            def _():
                ring_rdma(2, 0).wait_send()
            partial_into_send(2, (my_y - 3) % N_Y)
            rd1.wait_recv()

            accumulate_send(2, 1)
            signal_credit(1)

            pl.semaphore_wait(credit_sems.at[0], 1)
            rd2 = ring_rdma(2, 0)
            rd2.start()

            rd1.wait_send()
            partial_into_send(1, my_y)

            rd0.wait_send()

            @pl.when(eb < N_EB - 1)
            def _():
                load_w(eb + 1)
                partial_into_send(0, (my_y - 1) % N_Y)

            rd2.wait_recv()

            for h in range(2):
                rows = pl.ds(h * RH, RH)
                outstage_ref[rows] = (
                    send_ref[1, rows].astype(jnp.float32)
                    + comm_ref[0, rows].astype(jnp.float32))
            signal_credit(0)
            cpo = pltpu.make_async_copy(
                outstage_ref, out_ref.at[:, ecols], o_sem)
            cpo.start()
            cpo.wait()
            return carry

        lax.fori_loop(0, N_EB, eb_body, 0)

        ring_rdma(2, 0).wait_send()
        pl.semaphore_wait(credit_sems.at[0], 1)
        pl.semaphore_wait(credit_sems.at[1], 1)

    flat = pl.pallas_call(
        body,
        out_shape=jax.ShapeDtypeStruct((R, E), jnp.float32),
        in_specs=[pl.BlockSpec(memory_space=pl.ANY),
                  pl.BlockSpec(memory_space=pl.ANY)],
        out_specs=pl.BlockSpec(memory_space=pl.ANY),
        scratch_shapes=[
            pltpu.VMEM((3, R, EB), jnp.bfloat16),
            pltpu.VMEM((2, R, EB), jnp.bfloat16),
            pltpu.VMEM((R, K), jnp.bfloat16),
            pltpu.VMEM((K, EB), jnp.bfloat16),
            pltpu.VMEM((K, EB), jnp.float32),
            pltpu.VMEM((R, EB), jnp.float32),
            pltpu.SemaphoreType.DMA((3,)),
            pltpu.SemaphoreType.DMA((2,)),
            pltpu.SemaphoreType.REGULAR((2,)),
            pltpu.SemaphoreType.DMA,
            pltpu.SemaphoreType.DMA,
        ],
        compiler_params=pltpu.CompilerParams(
            vmem_limit_bytes=64 * 1024 * 1024,
        ),
    )(Oc, Wo)
    return flat.reshape(B, SC, E)


# device time: 1324142 ns/iter; 1.0176x vs baseline; 1.0176x over previous
import jax
import jax.numpy as jnp
from jax import lax
from jax.experimental import pallas as pl
from jax.experimental.pallas import tpu as pltpu

N_Y = 4
SC = 512
EB = 1024
N_EB = 8
R = 2048
RH = R // 2


def kernel(O, Wo):
    B, S, Hs, D = O.shape
    K = Hs * D
    E = Wo.shape[1]

    O16 = O.reshape(B, S, K).astype(jnp.bfloat16)
    Oc = O16.reshape(B, N_Y, SC, K).swapaxes(0, 1).reshape(N_Y, B * SC, K)

    def body(oc_ref, w_ref, out_ref, send_ref, comm_ref,
             ostage_ref, wstage_ref, wstage32_ref,
             send_sems, recv_sems, credit_sems, o_sem, w_sem):
        my_x = lax.axis_index("x")
        my_y = lax.axis_index("y")
        my_z = lax.axis_index("z")
        right = (my_x, (my_y + 1) % N_Y, my_z)
        left = (my_x, (my_y - 1) % N_Y, my_z)

        def signal_credit(slot):
            pl.semaphore_signal(
                credit_sems.at[slot], inc=1,
                device_id=left, device_id_type=pl.DeviceIdType.MESH)

        def ring_rdma(sslot, cslot):
            return pltpu.make_async_remote_copy(
                src_ref=send_ref.at[sslot],
                dst_ref=comm_ref.at[cslot],
                send_sem=send_sems.at[sslot],
                recv_sem=recv_sems.at[cslot],
                device_id=right,
                device_id_type=pl.DeviceIdType.MESH,
            )

        def load_w(eb):
            cw = pltpu.make_async_copy(
                w_ref.at[:, pl.ds(eb * EB, EB)], wstage32_ref, w_sem)
            cw.start()
            cw.wait()
            for h in range(2):
                rows = pl.ds(h * RH, RH)
                wstage_ref[rows] = wstage32_ref[rows].astype(jnp.bfloat16)

        def partial_into_send(sslot, chunk):
            co = pltpu.make_async_copy(oc_ref.at[chunk], ostage_ref, o_sem)
            co.start()
            co.wait()
            for h in range(2):
                rows = pl.ds(h * RH, RH)
                send_ref[sslot, rows] = jnp.dot(
                    ostage_ref[rows, :], wstage_ref[...],
                    preferred_element_type=jnp.float32,
                ).astype(jnp.bfloat16)

        def accumulate_send(sslot, cslot):
            for h in range(2):
                rows = pl.ds(h * RH, RH)
                send_ref[sslot, rows] = (send_ref[sslot, rows]
                                         + comm_ref[cslot, rows])

        load_w(0)
        partial_into_send(0, (my_y - 1) % N_Y)

        def eb_body(eb, carry):
            ecols = pl.ds(eb * EB, EB)

            @pl.when(eb > 0)
            def _():
                pl.semaphore_wait(credit_sems.at[0], 1)
            rd0 = ring_rdma(0, 0)
            rd0.start()

            partial_into_send(1, (my_y - 2) % N_Y)
            rd0.wait_recv()

            accumulate_send(1, 0)
            signal_credit(0)

            @pl.when(eb > 0)
            def _():
                pl.semaphore_wait(credit_sems.at[1], 1)
            rd1 = ring_rdma(1, 1)
            rd1.start()

            @pl.when(eb > 0)
            def _():
                ring_rdma(2, 0).wait_send()
            partial_into_send(2, (my_y - 3) % N_Y)
            rd1.wait_recv()

            accumulate_send(2, 1)
            signal_credit(1)

            pl.semaphore_wait(credit_sems.at[0], 1)
            rd2 = ring_rdma(2, 0)
            rd2.start()

            rd1.wait_send()
            partial_into_send(1, my_y)

            rd0.wait_send()

            @pl.when(eb < N_EB - 1)
            def _():
                load_w(eb + 1)
                partial_into_send(0, (my_y - 1) % N_Y)

            rd2.wait_recv()

            accumulate_send(1, 0)
            signal_credit(0)
            cpo = pltpu.make_async_copy(
                send_ref.at[1], out_ref.at[:, ecols], o_sem)
            cpo.start()
            cpo.wait()
            return carry

        lax.fori_loop(0, N_EB, eb_body, 0)

        ring_rdma(2, 0).wait_send()
        pl.semaphore_wait(credit_sems.at[0], 1)
        pl.semaphore_wait(credit_sems.at[1], 1)

    flat = pl.pallas_call(
        body,
        out_shape=jax.ShapeDtypeStruct((R, E), jnp.bfloat16),
        in_specs=[pl.BlockSpec(memory_space=pl.ANY),
                  pl.BlockSpec(memory_space=pl.ANY)],
        out_specs=pl.BlockSpec(memory_space=pl.ANY),
        scratch_shapes=[
            pltpu.VMEM((3, R, EB), jnp.bfloat16),
            pltpu.VMEM((2, R, EB), jnp.bfloat16),
            pltpu.VMEM((R, K), jnp.bfloat16),
            pltpu.VMEM((K, EB), jnp.bfloat16),
            pltpu.VMEM((K, EB), jnp.float32),
            pltpu.SemaphoreType.DMA((3,)),
            pltpu.SemaphoreType.DMA((2,)),
            pltpu.SemaphoreType.REGULAR((2,)),
            pltpu.SemaphoreType.DMA,
            pltpu.SemaphoreType.DMA,
        ],
        compiler_params=pltpu.CompilerParams(
            vmem_limit_bytes=64 * 1024 * 1024,
        ),
    )(Oc, Wo)
    return flat.astype(jnp.float32).reshape(B, SC, E)
